# baseline (speedup 1.0000x reference)
import functools

import jax
import jax.numpy as jnp
from jax import lax
from jax.experimental import pallas as pl
from jax.experimental.pallas import tpu as pltpu
from jax.experimental.pallas import tpu_sc as plsc

SEG = 32
LANES_OUT = 256
NW = 32
CHUNK = 160


def _sc_relayout(n_rows, in_dim, s):
    n_chunks = s // CHUNK
    per_w = -(-n_chunks // NW)
    rows_chunk = CHUNK * SEG
    flat = SEG * 16

    mesh = plsc.VectorSubcoreMesh(core_axis_name="c", subcore_axis_name="s")

    @functools.partial(
        pl.kernel, mesh=mesh,
        out_type=jax.ShapeDtypeStruct((s, flat), jnp.float32),
        compiler_params=pltpu.CompilerParams(use_tc_tiling_on_sc=False),
        scratch_types=[
            pltpu.VMEM((CHUNK, flat), jnp.float32),
            pltpu.SemaphoreType.DMA,
        ],
    )
    def k(data_hbm, out_hbm, vout, sem):
        sid = lax.axis_index("s")
        wid = sid * 2 + lax.axis_index("c")
        zeros16 = jnp.zeros((16,), jnp.float32)

        def zinit(p, carry):
            for j in range(flat // 16):
                vout[p, pl.ds(16 * j, 16)] = zeros16
            return carry
        lax.fori_loop(0, CHUNK, zinit, 0)

        def do_chunk(i, carry):
            ci = wid + i * NW

            @pl.when(ci < n_chunks)
            def _():
                p0 = ci * CHUNK

                def seg(p, c2):
                    row0 = (p0 + p) * SEG
                    cps = [
                        pltpu.make_async_copy(
                            data_hbm.at[row0 + c, :],
                            vout.at[p, pl.ds(c * 16, in_dim)],
                            sem)
                        for c in range(SEG)
                    ]
                    for cp in cps:
                        cp.start()
                    for cp in cps:
                        cp.wait()
                    return c2
                lax.fori_loop(0, CHUNK, seg, 0)
                pltpu.sync_copy(vout, out_hbm.at[pl.ds(p0, CHUNK), :])

            return carry

        lax.fori_loop(0, per_w, do_chunk, 0)

    return k


def _tc_body(x_ref, w1_ref, b1_ref, w2_ref, b2_ref, o_ref):
    x = x_ref[...].astype(jnp.bfloat16)
    h = jnp.dot(x, w1_ref[...], preferred_element_type=jnp.float32)
    h = jnp.maximum(h + b1_ref[...], 0).astype(jnp.bfloat16)
    w2 = w2_ref[...]
    y = None
    for o in range(4):
        part = jnp.dot(h[:, 128 * o:128 * (o + 1)], w2,
                       preferred_element_type=jnp.float32)
        y = part if y is None else jnp.maximum(y, part)
    v = jnp.maximum(y, jnp.roll(y, 64, axis=1))
    v = jnp.maximum(v, jnp.roll(v, 32, axis=1))
    v = jnp.maximum(v, jnp.roll(v, 16, axis=1))
    o_ref[...] = v[:, :16] + b2_ref[...]


def kernel(data, sizes, W1, b1, W2, b2):
    n, in_dim = data.shape
    s = sizes.shape[0]
    mid_dim = W1.shape[1]
    out_dim = W2.shape[1]

    data_f = _sc_relayout(n, in_dim, s)(data)

    w1p = jnp.concatenate([W1, jnp.zeros((11, mid_dim), jnp.float32)], axis=0)
    w1f = jnp.kron(jnp.eye(SEG, dtype=jnp.float32), w1p).astype(jnp.bfloat16)
    w2f = jnp.kron(jnp.eye(8, dtype=jnp.float32), W2).astype(jnp.bfloat16)
    b1f = jnp.tile(b1, SEG).reshape(1, -1)
    b2f = b2.reshape(1, -1)

    r = 2000
    grid = (s // r,)

    return pl.pallas_call(
        _tc_body,
        grid=grid,
        in_specs=[
            pl.BlockSpec((r, SEG * 16), lambda i: (i, 0)),
            pl.BlockSpec(w1f.shape, lambda i: (0, 0)),
            pl.BlockSpec((1, SEG * mid_dim), lambda i: (0, 0)),
            pl.BlockSpec(w2f.shape, lambda i: (0, 0)),
            pl.BlockSpec((1, out_dim), lambda i: (0, 0)),
        ],
        out_specs=pl.BlockSpec((r, out_dim), lambda i: (i, 0)),
        out_shape=jax.ShapeDtypeStruct((s, out_dim), jnp.float32),
        compiler_params=pltpu.CompilerParams(
            dimension_semantics=("arbitrary",),
        ),
    )(data_f, w1f, b1f, w2f, b2f)


# speedup vs baseline: 1.2116x; 1.2116x over previous
import functools

import jax
import jax.numpy as jnp
from jax import lax
from jax.experimental import pallas as pl
from jax.experimental.pallas import tpu as pltpu
from jax.experimental.pallas import tpu_sc as plsc

SEG = 32
GRP = 16
NW = 32


def _sc_relayout(n_rows, in_dim, s):
    n_pairs = s // 2
    per_w = -(-n_pairs // NW)

    mesh = plsc.VectorSubcoreMesh(core_axis_name="c", subcore_axis_name="s")

    @functools.partial(
        pl.kernel, mesh=mesh,
        out_type=jax.ShapeDtypeStruct((s // GRP, SEG, 8 * GRP), jnp.float32),
        compiler_params=pltpu.CompilerParams(use_tc_tiling_on_sc=False),
        scratch_types=[
            pltpu.VMEM((SEG, 16), jnp.float32),
            pltpu.SemaphoreType.DMA,
        ],
    )
    def k(data_hbm, out_hbm, tseg, sem):
        sid = lax.axis_index("s")
        wid = sid * 2 + lax.axis_index("c")
        zeros16 = jnp.zeros((16,), jnp.float32)

        def zinit(c, carry):
            tseg[c, pl.ds(0, 16)] = zeros16
            return carry
        lax.fori_loop(0, SEG, zinit, 0)

        def do_pair(i, carry):
            q = wid + i * NW

            @pl.when(q < n_pairs)
            def _():
                g = q // 8
                tp = q % 8
                cp0 = pltpu.make_async_copy(
                    data_hbm.at[pl.ds(q * 2 * SEG, SEG), :],
                    tseg.at[:, pl.ds(0, in_dim)], sem)
                cp1 = pltpu.make_async_copy(
                    data_hbm.at[pl.ds((q * 2 + 1) * SEG, SEG), :],
                    tseg.at[:, pl.ds(8, in_dim)], sem)
                cp0.start()
                cp1.start()
                cp0.wait()
                cp1.wait()
                pltpu.sync_copy(tseg, out_hbm.at[g, :, pl.ds(16 * tp, 16)])

            return carry

        lax.fori_loop(0, per_w, do_pair, 0)

    return k


def _tc_body(x_ref, w1_ref, b1_ref, w2_ref, b2_ref, o_ref):
    gb = x_ref.shape[0]
    x = x_ref[...].reshape(gb * SEG, 8 * GRP).astype(jnp.bfloat16)
    h = jnp.dot(x, w1_ref[...], preferred_element_type=jnp.float32)
    h = jnp.maximum(h + b1_ref[...], 0).astype(jnp.bfloat16)
    w2 = w2_ref[...]
    y_a = jnp.dot(h[:, :128], w2, preferred_element_type=jnp.float32)
    y_b = jnp.dot(h[:, 128:], w2, preferred_element_type=jnp.float32)
    y = jnp.concatenate([y_a, y_b], axis=1)
    v = jnp.max(y.reshape(gb, SEG, 16 * GRP), axis=1)
    o_ref[...] = (v + b2_ref[...]).reshape(gb, 1, 16 * GRP)


def kernel(data, sizes, W1, b1, W2, b2):
    n, in_dim = data.shape
    s = sizes.shape[0]
    mid_dim = W1.shape[1]
    out_dim = W2.shape[1]

    data_f = _sc_relayout(n, in_dim, s)(data)

    w1p = jnp.concatenate([W1, jnp.zeros((8 - in_dim, mid_dim),
                                         jnp.float32)], axis=0)
    w1f = jnp.kron(jnp.eye(GRP, dtype=jnp.float32), w1p).astype(jnp.bfloat16)
    w2f = jnp.kron(jnp.eye(8, dtype=jnp.float32), W2).astype(jnp.bfloat16)
    b1f = jnp.tile(b1, GRP).reshape(1, -1)
    b2f = jnp.tile(b2, GRP).reshape(1, -1)

    gblk = 125
    grid = ((s // GRP) // gblk,)

    out5 = pl.pallas_call(
        _tc_body,
        grid=grid,
        in_specs=[
            pl.BlockSpec((gblk, SEG, 8 * GRP), lambda i: (i, 0, 0)),
            pl.BlockSpec(w1f.shape, lambda i: (0, 0)),
            pl.BlockSpec((1, GRP * mid_dim), lambda i: (0, 0)),
            pl.BlockSpec(w2f.shape, lambda i: (0, 0)),
            pl.BlockSpec((1, GRP * out_dim), lambda i: (0, 0)),
        ],
        out_specs=pl.BlockSpec((gblk, 1, GRP * out_dim),
                               lambda i: (i, 0, 0)),
        out_shape=jax.ShapeDtypeStruct((s // GRP, 1, GRP * out_dim),
                                       jnp.float32),
        compiler_params=pltpu.CompilerParams(
            dimension_semantics=("arbitrary",),
        ),
    )(data_f, w1f, b1f, w2f, b2f)
    return out5.reshape(s, out_dim)


# speedup vs baseline: 1.4904x; 1.2301x over previous
import functools

import jax
import jax.numpy as jnp
from jax import lax
from jax.experimental import pallas as pl
from jax.experimental.pallas import tpu as pltpu
from jax.experimental.pallas import tpu_sc as plsc

SEG = 32
GRP = 16
NW = 32


def _sc_relayout(n_rows, in_dim, s):
    n_pairs = s // 2
    per_w = -(-n_pairs // NW)

    mesh = plsc.VectorSubcoreMesh(core_axis_name="c", subcore_axis_name="s")

    @functools.partial(
        pl.kernel, mesh=mesh,
        out_type=jax.ShapeDtypeStruct((s // GRP, SEG, 8 * GRP), jnp.float32),
        compiler_params=pltpu.CompilerParams(use_tc_tiling_on_sc=False),
        scratch_types=[
            pltpu.VMEM((8, SEG, 16), jnp.float32),
            pltpu.SemaphoreType.DMA,
            pltpu.SemaphoreType.DMA,
        ],
    )
    def k(data_hbm, out_hbm, tseg, sem_in, sem_out):
        sid = lax.axis_index("s")
        wid = sid * 2 + lax.axis_index("c")
        zeros16 = jnp.zeros((16,), jnp.float32)

        def zinit(c, carry):
            for b in range(8):
                tseg[b, c, pl.ds(0, 16)] = zeros16
            return carry
        lax.fori_loop(0, SEG, zinit, 0)

        n_rounds = -(-per_w // 8)

        def do_round(ib, carry):
            qs = [wid + (ib * 8 + b) * NW for b in range(8)]
            cps = []
            for b in range(8):
                q = qs[b]
                cp0 = pltpu.make_async_copy(
                    data_hbm.at[pl.ds(q * 2 * SEG, SEG), :],
                    tseg.at[b, :, pl.ds(0, in_dim)], sem_in)
                cp1 = pltpu.make_async_copy(
                    data_hbm.at[pl.ds((q * 2 + 1) * SEG, SEG), :],
                    tseg.at[b, :, pl.ds(8, in_dim)], sem_in)
                cps.append((cp0, cp1))

                @pl.when(q < n_pairs)
                def _():
                    cp0.start()
                    cp1.start()
            outs = []
            for b in range(8):
                q = qs[b]
                cp0, cp1 = cps[b]
                po = pltpu.make_async_copy(
                    tseg.at[b],
                    out_hbm.at[q // 8, :, pl.ds(16 * (q % 8), 16)], sem_out)
                outs.append(po)

                @pl.when(q < n_pairs)
                def _():
                    cp0.wait()
                    cp1.wait()
                    po.start()
            for b in range(8):
                q = qs[b]
                po = outs[b]

                @pl.when(q < n_pairs)
                def _():
                    po.wait()
            return carry

        lax.fori_loop(0, n_rounds, do_round, 0)

    return k


def _tc_body(x_ref, w1_ref, b1_ref, w2_ref, b2_ref, o_ref):
    gb = x_ref.shape[0]
    x = x_ref[...].reshape(gb * SEG, 8 * GRP).astype(jnp.bfloat16)
    h = jnp.dot(x, w1_ref[...], preferred_element_type=jnp.float32)
    h = jnp.maximum(h + b1_ref[...], 0).astype(jnp.bfloat16)
    w2 = w2_ref[...]
    y_a = jnp.dot(h[:, :128], w2, preferred_element_type=jnp.float32)
    y_b = jnp.dot(h[:, 128:], w2, preferred_element_type=jnp.float32)
    y = jnp.concatenate([y_a, y_b], axis=1)
    v = jnp.max(y.reshape(gb, SEG, 16 * GRP), axis=1)
    o_ref[...] = (v + b2_ref[...]).reshape(gb, 1, 16 * GRP)


def kernel(data, sizes, W1, b1, W2, b2):
    n, in_dim = data.shape
    s = sizes.shape[0]
    mid_dim = W1.shape[1]
    out_dim = W2.shape[1]

    data_f = _sc_relayout(n, in_dim, s)(data)

    w1p = jnp.concatenate([W1, jnp.zeros((8 - in_dim, mid_dim),
                                         jnp.float32)], axis=0)
    w1f = jnp.kron(jnp.eye(GRP, dtype=jnp.float32), w1p).astype(jnp.bfloat16)
    w2f = jnp.kron(jnp.eye(8, dtype=jnp.float32), W2).astype(jnp.bfloat16)
    b1f = jnp.tile(b1, GRP).reshape(1, -1)
    b2f = jnp.tile(b2, GRP).reshape(1, -1)

    gblk = 125
    grid = ((s // GRP) // gblk,)

    out5 = pl.pallas_call(
        _tc_body,
        grid=grid,
        in_specs=[
            pl.BlockSpec((gblk, SEG, 8 * GRP), lambda i: (i, 0, 0)),
            pl.BlockSpec(w1f.shape, lambda i: (0, 0)),
            pl.BlockSpec((1, GRP * mid_dim), lambda i: (0, 0)),
            pl.BlockSpec(w2f.shape, lambda i: (0, 0)),
            pl.BlockSpec((1, GRP * out_dim), lambda i: (0, 0)),
        ],
        out_specs=pl.BlockSpec((gblk, 1, GRP * out_dim),
                               lambda i: (i, 0, 0)),
        out_shape=jax.ShapeDtypeStruct((s // GRP, 1, GRP * out_dim),
                                       jnp.float32),
        compiler_params=pltpu.CompilerParams(
            dimension_semantics=("arbitrary",),
        ),
    )(data_f, w1f, b1f, w2f, b2f)
    return out5.reshape(s, out_dim)
